# Initial kernel scaffold; baseline (speedup 1.0000x reference)
import jax
import jax.numpy as jnp
from jax.experimental import pallas as pl


def kernel(x_paper, x_author, edge_index_writes, edge_index_written_by, Wself_paper, bself_paper, Wself_author, bself_author, Wq_paper, bq_paper, Wk_paper, bk_paper, Wq_author, bq_author, Wk_author, bk_author, Wal_paper, bal_paper, Wal_author, bal_author, War_paper, bar_paper, War_author, bar_author, Wconv_writes, bconv_writes, Wconv_written_by, bconv_written_by, Wcls, bcls):
    raise NotImplementedError("write your pallas kernel here")


# speedup vs baseline: 2.8034x; 2.8034x over previous
import functools

import jax
import jax.numpy as jnp
from jax import lax
from jax.experimental import pallas as pl
from jax.experimental.pallas import tpu as pltpu
from jax.experimental.pallas import tpu_sc as plsc

N_PAPER = 50000
N_AUTHOR = 50000
E = 300000
D = 128

NP_PAD = 50176
NR = 8
RANGE = NP_PAD // NR
TRASH = RANGE
EP_PAD = 311296
EDGE_ROWS = EP_PAD // 128
ROWS_PER_TILE = EDGE_ROWS // 16
SCAN_ITERS = ROWS_PER_TILE * 8
SP_ROWS = 6400
DEG_BINS = 7168


def _sc_body(src_hbm, dst_hbm, x_hbm, agg_hbm, deg_hbm,
             src_v, dst_v, rows_v, zbuf, zero1, ones_v, cnt_v,
             sp_agg, sp_deg, sem):
    cid = lax.axis_index("c")
    sid = lax.axis_index("s")
    t0 = sid * ROWS_PER_TILE
    iota = lax.iota(jnp.int32, 16)
    zero16i = jnp.zeros((16,), jnp.int32)
    one16i = jnp.full((16,), 1, jnp.int32)

    for k in range(8):
        ones_v[pl.ds(16 * k, 16)] = jnp.ones((16,), jnp.float32)

    def _z2(i, _):
        r = i // 8
        c = (i % 8) * 16
        zbuf[r, pl.ds(c, 16)] = jnp.zeros((16,), jnp.float32)
        return 0
    lax.fori_loop(0, 1024, _z2, 0)

    def _z1(i, _):
        zero1[pl.ds(i * 16, 16)] = jnp.zeros((16,), jnp.float32)
        return 0
    lax.fori_loop(0, (DEG_BINS // 16) // 16, _z1, 0)

    for p in range(4):
        base = (4 * cid + p) * RANGE

        z0 = sid * (SP_ROWS // 16)
        for k in range(3):
            pltpu.sync_copy(zbuf, sp_agg.at[pl.ds(z0 + 128 * k, 128)])
        pltpu.sync_copy(zbuf.at[pl.ds(0, 16)], sp_agg.at[pl.ds(z0 + 384, 16)])
        pltpu.sync_copy(zero1, sp_deg.at[pl.ds(sid * (DEG_BINS // 16),
                                               DEG_BINS // 16)])
        plsc.subcore_barrier()

        pltpu.sync_copy(src_hbm.at[pl.ds(t0, ROWS_PER_TILE)], src_v)
        pltpu.sync_copy(dst_hbm.at[pl.ds(t0, ROWS_PER_TILE)], dst_v)

        basev = jnp.full((16,), base, jnp.int32)
        topv = jnp.full((16,), base + RANGE, jnp.int32)
        cnt_v[pl.ds(0, 16)] = zero16i

        def _scan(i, _):
            r = i // 8
            c = (i % 8) * 16
            s16 = src_v[r, pl.ds(c, 16)]
            d16 = dst_v[r, pl.ds(c, 16)]
            loc = d16 - basev
            mask = (d16 >= basev) & (d16 < topv)
            mi = jnp.where(mask, one16i, zero16i)
            nv = cnt_v[pl.ds(0, 16)]
            pos = nv + plsc.cumsum(mi) - 1
            plsc.store_scatter(src_v, [pos >> 7, pos & 127], s16, mask=mask)
            plsc.store_scatter(dst_v, [pos >> 7, pos & 127], loc, mask=mask)
            cnt_v[pl.ds(0, 16)] = nv + plsc.all_reduce_population_count(mask)
            return 0

        lax.fori_loop(0, SCAN_ITERS, _scan, 0)

        nv = cnt_v[pl.ds(0, 16)]
        nbv = (nv + 127) >> 7
        endv = nbv * 128
        for k in range(8):
            pos = nv + 16 * k + iota
            m = pos < endv
            plsc.store_scatter(src_v, [pos >> 7, pos & 127], zero16i, mask=m)
            plsc.store_scatter(dst_v, [pos >> 7, pos & 127],
                               jnp.full((16,), TRASH, jnp.int32), mask=m)

        def _cond(j):
            nb2 = (cnt_v[pl.ds(0, 16)] + 127) >> 7
            return jnp.any(jnp.full((16,), j, jnp.int32) < nb2)

        def _blk(j):
            pltpu.async_copy(x_hbm.at[src_v.at[j]], rows_v, sem).wait()
            pltpu.sync_copy(rows_v, sp_agg.at[dst_v.at[j]], add=True)
            pltpu.sync_copy(ones_v, sp_deg.at[dst_v.at[j]], add=True)
            return j + 1

        lax.while_loop(_cond, _blk, jnp.int32(0))
        plsc.subcore_barrier()

        o0 = sid * (RANGE // 16)
        pltpu.sync_copy(sp_agg.at[pl.ds(o0, RANGE // 16)],
                        agg_hbm.at[pl.ds(base + o0, RANGE // 16)])

        @pl.when(sid == 0)
        def _():
            pltpu.sync_copy(sp_deg.at[pl.ds(0, RANGE)],
                            deg_hbm.at[pl.ds(base, RANGE)])
        plsc.subcore_barrier()


_sc_segsum = functools.partial(
    pl.kernel,
    out_type=[
        jax.ShapeDtypeStruct((NP_PAD, D), jnp.float32),
        jax.ShapeDtypeStruct((NP_PAD,), jnp.float32),
    ],
    mesh=plsc.VectorSubcoreMesh(core_axis_name="c", subcore_axis_name="s"),
    compiler_params=pltpu.CompilerParams(needs_layout_passes=False),
    scratch_types=[
        pltpu.VMEM((ROWS_PER_TILE, 128), jnp.int32),
        pltpu.VMEM((ROWS_PER_TILE, 128), jnp.int32),
        pltpu.VMEM((128, D), jnp.float32),
        pltpu.VMEM((128, D), jnp.float32),
        pltpu.VMEM((DEG_BINS // 16,), jnp.float32),
        pltpu.VMEM((128,), jnp.float32),
        pltpu.VMEM((16,), jnp.int32),
        pltpu.VMEM_SHARED((SP_ROWS, D), jnp.float32),
        pltpu.VMEM_SHARED((DEG_BINS,), jnp.float32),
        pltpu.SemaphoreType.DMA,
    ],
)(_sc_body)


def _elu(x):
    return jnp.where(x > 0, x, jnp.exp(x) - 1.0)


def _tc_body(x_ref, agg_ref, deg_ref, wself_ref, bself_ref, wconv_ref,
             bconv_ref, wk_ref, bk_ref, wq_ref, bq_ref, walT_ref, bal_ref,
             warT_ref, bar_ref, wcls_ref, bcls_ref, out_ref):
    f32 = jnp.float32

    def dot(a, b):
        return jnp.dot(a, b, preferred_element_type=f32)

    def dotT(a, b):
        return lax.dot_general(a, b, (((1,), (1,)), ((), ())),
                               preferred_element_type=f32)

    x = x_ref[...]
    ag = agg_ref[...]
    degrow = deg_ref[0]
    z = dot(x, wself_ref[...]) + bself_ref[...]
    conv = dot(ag, wconv_ref[...])
    rdeg = 1.0 / jnp.maximum(degrow, 1.0)
    ri = lax.broadcasted_iota(jnp.int32, (128, 128), 0)
    ci = lax.broadcasted_iota(jnp.int32, (128, 128), 1)
    eye = (ri == ci).astype(f32)
    rdeg_col = dotT(eye, rdeg)
    d = conv * rdeg_col + bconv_ref[...]
    u1 = dotT(wk_ref[...], walT_ref[...])
    c1 = jnp.sum(bk_ref[...] * walT_ref[...]) + bal_ref[0, 0]
    u2 = dotT(wq_ref[...], warT_ref[...])
    c2 = jnp.sum(bq_ref[...] * warT_ref[...]) + bar_ref[0, 0]
    hr = dot(z, u2) + c2
    a0 = _elu(dot(z, u1) + c1 + hr)
    a1 = _elu(dot(d, u1) + c1 + hr)
    mx = jnp.maximum(a0, a1)
    e0 = jnp.exp(a0 - mx)
    e1 = jnp.exp(a1 - mx)
    inv = 1.0 / (e0 + e1)
    rst = z * (e0 * inv) + d * (e1 * inv)
    out_ref[...] = dot(rst, wcls_ref[...]) + bcls_ref[...]


def _tc_fused(xp, agg, deg3d, wself, bself, wconv, bconv, wk, bk, wq, bq,
              walT, bal, warT, bar, wcls, bcls):
    nblk = NP_PAD // 128
    full = lambda shape: pl.BlockSpec(shape, lambda i: (0, 0))
    return pl.pallas_call(
        _tc_body,
        grid=(nblk,),
        in_specs=[
            pl.BlockSpec((128, D), lambda i: (i, 0)),
            pl.BlockSpec((128, D), lambda i: (i, 0)),
            pl.BlockSpec((1, 1, 128), lambda i: (i, 0, 0)),
            full((D, D)), full((1, D)), full((D, D)), full((1, D)),
            full((D, D)), full((1, D)), full((D, D)), full((1, D)),
            full((1, D)),
            pl.BlockSpec(memory_space=pltpu.SMEM),
            full((1, D)),
            pl.BlockSpec(memory_space=pltpu.SMEM),
            full((D, 16)), full((1, 16)),
        ],
        out_specs=pl.BlockSpec((128, 16), lambda i: (i, 0)),
        out_shape=jax.ShapeDtypeStruct((NP_PAD, 16), jnp.float32),
    )(xp, agg, deg3d, wself, bself, wconv, bconv, wk, bk, wq, bq,
      walT, bal, warT, bar, wcls, bcls)


def kernel(x_paper, x_author, edge_index_writes, edge_index_written_by,
           Wself_paper, bself_paper, Wself_author, bself_author,
           Wq_paper, bq_paper, Wk_paper, bk_paper,
           Wq_author, bq_author, Wk_author, bk_author,
           Wal_paper, bal_paper, Wal_author, bal_author,
           War_paper, bar_paper, War_author, bar_author,
           Wconv_writes, bconv_writes, Wconv_written_by, bconv_written_by,
           Wcls, bcls):
    xp = jnp.pad(x_paper, ((0, NP_PAD - N_PAPER), (0, 0)))
    src = jnp.pad(edge_index_writes[0], (0, EP_PAD - E)).reshape(EDGE_ROWS, 128)
    dst = jnp.pad(edge_index_writes[1], (0, EP_PAD - E),
                  constant_values=jnp.int32(1 << 30)).reshape(EDGE_ROWS, 128)
    agg, deg = _sc_segsum(src, dst, x_author)
    logits = _tc_fused(
        xp, agg, deg.reshape(NP_PAD // 128, 1, 128),
        Wself_paper, bself_paper.reshape(1, D),
        Wconv_writes, bconv_writes.reshape(1, D),
        Wk_paper, bk_paper.reshape(1, D),
        Wq_paper, bq_paper.reshape(1, D),
        Wal_paper.reshape(1, D), bal_paper.reshape(1, 1),
        War_paper.reshape(1, D), bar_paper.reshape(1, 1),
        Wcls, bcls.reshape(1, 16))
    return logits[:N_PAPER]
